# baseline (speedup 1.0000x reference)
import functools

import jax
import jax.numpy as jnp
from jax import lax
from jax.experimental import pallas as pl
from jax.experimental.pallas import tpu as pltpu
from jax.experimental.pallas import tpu_sc as plsc

B = 16384
K = 20
D = 64
SPB = 32
HALF = 16
NEG_PER_BLOCK = SPB * K
NBLK = B // SPB
NW = 32
BLOCKS_PER_W = NBLK // NW


def _sc_scores(centers_r, pos_r, neg_r, W_in, W_out):
    mesh = plsc.VectorSubcoreMesh(core_axis_name="c", subcore_axis_name="s")

    @functools.partial(
        pl.kernel,
        mesh=mesh,
        compiler_params=pltpu.CompilerParams(
            needs_layout_passes=False, use_tc_tiling_on_sc=False),
        out_type=[
            jax.ShapeDtypeStruct((NBLK, SPB), jnp.float32),
            jax.ShapeDtypeStruct((NBLK, NEG_PER_BLOCK), jnp.float32),
        ],
        scratch_types=[
            pltpu.VMEM((SPB,), jnp.int32),
            pltpu.VMEM((SPB,), jnp.int32),
            pltpu.VMEM((5, 128), jnp.int32),
            pltpu.VMEM((SPB, D), jnp.float32),
            pltpu.VMEM((SPB, D), jnp.float32),
            pltpu.VMEM((NEG_PER_BLOCK, D), jnp.float32),
            pltpu.VMEM((SPB,), jnp.float32),
            pltpu.VMEM((NEG_PER_BLOCK,), jnp.float32),
            pltpu.SemaphoreType.DMA,
        ],
    )
    def k(centers_h, pos_h, neg_h, W_in_h, W_out_h, psc_h, nsc_h,
          cidx_v, pidx_v, nidx_v, vc_v, uo_v, un_v, psc_v, nsc_v, sem):
        wid = lax.axis_index("s") * 2 + lax.axis_index("c")

        def do_block(i, carry):
            blk = wid * BLOCKS_PER_W + i
            pltpu.sync_copy(centers_h.at[blk], cidx_v)
            pltpu.sync_copy(pos_h.at[blk], pidx_v)
            pltpu.sync_copy(neg_h.at[blk], nidx_v)
            copies = [
                pltpu.async_copy(W_in_h.at[cidx_v], vc_v, sem),
                pltpu.async_copy(W_out_h.at[pidx_v], uo_v, sem),
            ]
            for c in range(5):
                copies.append(
                    pltpu.async_copy(W_out_h.at[nidx_v.at[c]],
                                     un_v.at[pl.ds(c * 128, 128)], sem))
            for cp in copies:
                cp.wait()

            lanes = lax.iota(jnp.int32, HALF)
            for h in range(SPB // HALF):
                srows = lanes + (h * HALF)
                krows = [lanes * K + (h * HALF * K + kk) for kk in range(K)]

                def dstep(d, acc):
                    dcol = jnp.full((HALF,), 0, jnp.int32) + d
                    vcol = plsc.load_gather(vc_v, [srows, dcol])
                    pcol = plsc.load_gather(uo_v, [srows, dcol])
                    new = [acc[0] + vcol * pcol]
                    for kk in range(K):
                        ncol = plsc.load_gather(un_v, [krows[kk], dcol])
                        new.append(acc[kk + 1] + vcol * ncol)
                    return tuple(new)

                acc0 = (jnp.zeros((HALF,), jnp.float32),) * (K + 1)
                acc = lax.fori_loop(0, D, dstep, acc0)
                psc_v[pl.ds(h * HALF, HALF)] = acc[0]
                for kk in range(K):
                    nsc_v[pl.ds((h * HALF * K) + kk * HALF, HALF)] = acc[kk + 1]

            pltpu.sync_copy(psc_v, psc_h.at[blk])
            pltpu.sync_copy(nsc_v, nsc_h.at[blk])
            return carry

        lax.fori_loop(0, BLOCKS_PER_W, do_block, 0)

    return k(centers_r, pos_r, neg_r, W_in, W_out)


def _tc_loss_kernel(ps_ref, ns_ref, out_ref):
    def log_sig(x):
        return jnp.minimum(x, 0.0) - jnp.log1p(jnp.exp(-jnp.abs(x)))

    tot = jnp.sum(log_sig(ps_ref[...])) + jnp.sum(log_sig(-ns_ref[...]))
    out_ref[...] = jnp.full((1, 1), -tot / B, dtype=jnp.float32)


def kernel(centers, pos, neg, W_in, W_out):
    centers_r = centers.reshape(NBLK, SPB)
    pos_r = pos.reshape(NBLK, SPB)
    neg_r = neg.reshape(NBLK, 5, 128)

    psc, nsc = _sc_scores(centers_r, pos_r, neg_r, W_in, W_out)

    ps2 = psc.reshape(B // 128, 128)
    ns2 = nsc.reshape(B * K // 128, 128)
    loss = pl.pallas_call(
        _tc_loss_kernel,
        out_shape=jax.ShapeDtypeStruct((1, 1), jnp.float32),
    )(ps2, ns2)
    return loss.reshape(())


# speedup vs baseline: 1.3162x; 1.3162x over previous
import functools

import jax
import jax.numpy as jnp
from jax import lax
from jax.experimental import pallas as pl
from jax.experimental.pallas import tpu as pltpu
from jax.experimental.pallas import tpu_sc as plsc

B = 16384
K = 20
D = 64
VOCAB = 1000000
SPB = 32
HALF = 16
NEG_PER_BLOCK = SPB * K
NW = 32
BLOCKS_PER_W = B // SPB // NW
TCH = 4096


def _tr_body(i_ref, o_ref):
    o_ref[:, 0:D] = i_ref[...].T


def _transpose_table(Wt):
    grid = (VOCAB + TCH - 1) // TCH
    return pl.pallas_call(
        _tr_body,
        grid=(grid,),
        in_specs=[pl.BlockSpec((D, TCH), lambda j: (0, j))],
        out_specs=pl.BlockSpec((TCH, 128), lambda j: (j, 0)),
        out_shape=jax.ShapeDtypeStruct((VOCAB, 128), jnp.float32),
    )(Wt)


def _sc_scores(centers_r, pos_r, neg_r, W_in, W_out):
    mesh = plsc.VectorSubcoreMesh(core_axis_name="c", subcore_axis_name="s")

    @functools.partial(
        pl.kernel,
        mesh=mesh,
        compiler_params=pltpu.CompilerParams(
            needs_layout_passes=False, use_tc_tiling_on_sc=False),
        out_type=[
            jax.ShapeDtypeStruct((NW, BLOCKS_PER_W, SPB), jnp.float32),
            jax.ShapeDtypeStruct((NW, BLOCKS_PER_W * 5, 128), jnp.float32),
        ],
        scratch_types=[
            pltpu.VMEM((BLOCKS_PER_W, SPB), jnp.int32),
            pltpu.VMEM((BLOCKS_PER_W, SPB), jnp.int32),
            pltpu.VMEM((BLOCKS_PER_W * 5, 128), jnp.int32),
            pltpu.VMEM((SPB, 128), jnp.float32),
            pltpu.VMEM((SPB, 128), jnp.float32),
            pltpu.VMEM((NEG_PER_BLOCK, 128), jnp.float32),
            pltpu.VMEM((BLOCKS_PER_W, SPB), jnp.float32),
            pltpu.VMEM((BLOCKS_PER_W * 5, 128), jnp.float32),
            pltpu.SemaphoreType.DMA,
        ],
    )
    def k(centers_h, pos_h, neg_h, W_in_h, W_out_h, psc_h, nsc_h,
          cidx_v, pidx_v, nidx_v, vc_v, uo_v, un_v, psc_v, nsc_v, sem):
        wid = lax.axis_index("s") * 2 + lax.axis_index("c")

        pltpu.sync_copy(centers_h.at[wid], cidx_v)
        pltpu.sync_copy(pos_h.at[wid], pidx_v)
        pltpu.sync_copy(neg_h.at[wid], nidx_v)

        def do_block(i, carry):
            copies = [
                pltpu.async_copy(W_in_h.at[cidx_v.at[i]], vc_v, sem),
                pltpu.async_copy(W_out_h.at[pidx_v.at[i]], uo_v, sem),
            ]
            for c in range(5):
                copies.append(
                    pltpu.async_copy(W_out_h.at[nidx_v.at[i * 5 + c]],
                                     un_v.at[pl.ds(c * 128, 128)], sem))
            for cp in copies:
                cp.wait()

            lanes = lax.iota(jnp.int32, HALF)
            for h in range(SPB // HALF):
                srows = lanes + (h * HALF)
                krows = [lanes * K + (h * HALF * K + kk) for kk in range(K)]

                def dstep(d, acc):
                    dcol = jnp.full((HALF,), 0, jnp.int32) + d
                    vcol = plsc.load_gather(vc_v, [srows, dcol])
                    pcol = plsc.load_gather(uo_v, [srows, dcol])
                    new = [acc[0] + vcol * pcol]
                    for kk in range(K):
                        ncol = plsc.load_gather(un_v, [krows[kk], dcol])
                        new.append(acc[kk + 1] + vcol * ncol)
                    return tuple(new)

                acc0 = (jnp.zeros((HALF,), jnp.float32),) * (K + 1)
                acc = lax.fori_loop(0, D, dstep, acc0)
                psc_v[i, pl.ds(h * HALF, HALF)] = acc[0]
                for kk in range(K):
                    off = h * HALF * K + kk * HALF
                    nsc_v[i * 5 + off // 128, pl.ds(off % 128, HALF)] = (
                        acc[kk + 1])
            return carry

        lax.fori_loop(0, BLOCKS_PER_W, do_block, 0)

        pltpu.sync_copy(psc_v, psc_h.at[wid])
        pltpu.sync_copy(nsc_v, nsc_h.at[wid])

    return k(centers_r, pos_r, neg_r, W_in, W_out)


def _tc_loss_kernel(ps_ref, ns_ref, out_ref):
    def log_sig(x):
        return jnp.minimum(x, 0.0) - jnp.log1p(jnp.exp(-jnp.abs(x)))

    tot = jnp.sum(log_sig(ps_ref[...])) + jnp.sum(log_sig(-ns_ref[...]))
    out_ref[...] = jnp.full((1, 1), -tot / B, dtype=jnp.float32)


def kernel(centers, pos, neg, W_in, W_out):
    W_in_rm = _transpose_table(W_in.T)
    W_out_rm = _transpose_table(W_out.T)

    centers_r = centers.reshape(NW, BLOCKS_PER_W, SPB)
    pos_r = pos.reshape(NW, BLOCKS_PER_W, SPB)
    neg_r = neg.reshape(NW, BLOCKS_PER_W * 5, 128)

    psc, nsc = _sc_scores(centers_r, pos_r, neg_r, W_in_rm, W_out_rm)

    ps2 = psc.reshape(B // 128, 128)
    ns2 = nsc.reshape(B * K // 128, 128)
    loss = pl.pallas_call(
        _tc_loss_kernel,
        out_shape=jax.ShapeDtypeStruct((1, 1), jnp.float32),
    )(ps2, ns2)
    return loss.reshape(())


# speedup vs baseline: 1.6248x; 1.2344x over previous
import functools

import jax
import jax.numpy as jnp
from jax import lax
from jax.experimental import pallas as pl
from jax.experimental.pallas import tpu as pltpu
from jax.experimental.pallas import tpu_sc as plsc

B = 16384
K = 20
D = 64
VOCAB = 1000000
SPB = 16
NPB = SPB * K
NW = 32
BPW = B // SPB // NW
TCH = 4096


def _tr_body(a_ref, b_ref, oa_ref, ob_ref):
    oa_ref[:, 0:D] = a_ref[...].T
    ob_ref[:, 0:D] = b_ref[...].T


def _transpose_tables(Wa, Wb):
    grid = (VOCAB + TCH - 1) // TCH
    ispec = pl.BlockSpec((D, TCH), lambda j: (0, j))
    ospec = pl.BlockSpec((TCH, 128), lambda j: (j, 0))
    oshape = jax.ShapeDtypeStruct((VOCAB, 128), jnp.float32)
    return pl.pallas_call(
        _tr_body,
        grid=(grid,),
        in_specs=[ispec, ispec],
        out_specs=[ospec, ospec],
        out_shape=[oshape, oshape],
    )(Wa, Wb)


def _sc_scores(centers_r, pos_r, neg_r, W_in, W_out):
    mesh = plsc.VectorSubcoreMesh(core_axis_name="c", subcore_axis_name="s")

    rowbuf = [
        pltpu.VMEM((SPB, 128), jnp.float32),
        pltpu.VMEM((SPB, 128), jnp.float32),
        pltpu.VMEM((NPB, 128), jnp.float32),
        pltpu.SemaphoreType.DMA,
    ]

    @functools.partial(
        pl.kernel,
        mesh=mesh,
        compiler_params=pltpu.CompilerParams(
            needs_layout_passes=False, use_tc_tiling_on_sc=False),
        out_type=[
            jax.ShapeDtypeStruct((NW, BPW, SPB), jnp.float32),
            jax.ShapeDtypeStruct((NW, BPW * 5, D), jnp.float32),
        ],
        scratch_types=[
            pltpu.VMEM((BPW, SPB), jnp.int32),
            pltpu.VMEM((BPW, SPB), jnp.int32),
            pltpu.VMEM((BPW * 5, D), jnp.int32),
            pltpu.VMEM((BPW, SPB), jnp.float32),
            pltpu.VMEM((BPW * 5, D), jnp.float32),
        ] + rowbuf + rowbuf,
    )
    def k(centers_h, pos_h, neg_h, W_in_h, W_out_h, psc_h, nsc_h,
          cidx_v, pidx_v, nidx_v, psc_v, nsc_v,
          vc0, uo0, un0, sem0, vc1, uo1, un1, sem1):
        wid = lax.axis_index("s") * 2 + lax.axis_index("c")
        bufs = ((vc0, uo0, un0, sem0), (vc1, uo1, un1, sem1))

        pltpu.sync_copy(centers_h.at[wid], cidx_v)
        pltpu.sync_copy(pos_h.at[wid], pidx_v)
        pltpu.sync_copy(neg_h.at[wid], nidx_v)

        def issue(blk, b):
            vc_v, uo_v, un_v, sem = bufs[b]
            pltpu.async_copy(W_in_h.at[cidx_v.at[blk]], vc_v, sem)
            pltpu.async_copy(W_out_h.at[pidx_v.at[blk]], uo_v, sem)
            for c in range(5):
                pltpu.async_copy(W_out_h.at[nidx_v.at[blk * 5 + c]],
                                 un_v.at[pl.ds(c * D, D)], sem)

        def drain(b):
            vc_v, uo_v, un_v, sem = bufs[b]
            pltpu.make_async_copy(W_in_h.at[cidx_v.at[0]], vc_v, sem).wait()
            pltpu.make_async_copy(W_in_h.at[pidx_v.at[0]], uo_v, sem).wait()
            for c in range(5):
                pltpu.make_async_copy(W_out_h.at[nidx_v.at[c]],
                                      un_v.at[pl.ds(c * D, D)], sem).wait()

        lanes = lax.iota(jnp.int32, SPB)
        krows = [lanes * K + kk for kk in range(K)]

        def compute(blk, b):
            vc_v, uo_v, un_v, _ = bufs[b]

            def dstep(d, acc):
                dcol = jnp.full((SPB,), 0, jnp.int32) + d
                vcol = plsc.load_gather(vc_v, [lanes, dcol])
                pcol = plsc.load_gather(uo_v, [lanes, dcol])
                new = [acc[0] + vcol * pcol]
                for kk in range(K):
                    ncol = plsc.load_gather(un_v, [krows[kk], dcol])
                    new.append(acc[kk + 1] + vcol * ncol)
                return tuple(new)

            acc0 = (jnp.zeros((SPB,), jnp.float32),) * (K + 1)
            acc = lax.fori_loop(0, D, dstep, acc0)
            psc_v[blk, :] = acc[0]
            for kk in range(K):
                nsc_v[blk * 5 + (kk * SPB) // D,
                      pl.ds((kk * SPB) % D, SPB)] = acc[kk + 1]

        issue(0, 0)
        issue(1, 1)

        def ring(t, carry):
            for b in range(2):
                blk = t * 2 + b
                drain(b)
                compute(blk, b)
                issue(blk + 2, b)
            return carry

        lax.fori_loop(0, BPW // 2 - 1, ring, 0)
        for b in range(2):
            drain(b)
            compute(BPW - 2 + b, b)

        pltpu.sync_copy(psc_v, psc_h.at[wid])
        pltpu.sync_copy(nsc_v, nsc_h.at[wid])

    return k(centers_r, pos_r, neg_r, W_in, W_out)


def _tc_loss_kernel(ps_ref, ns_ref, out_ref):
    def log_sig(x):
        return jnp.minimum(x, 0.0) - jnp.log1p(jnp.exp(-jnp.abs(x)))

    tot = jnp.sum(log_sig(ps_ref[...])) + jnp.sum(log_sig(-ns_ref[...]))
    out_ref[...] = jnp.full((1, 1), -tot / B, dtype=jnp.float32)


def kernel(centers, pos, neg, W_in, W_out):
    W_in_rm, W_out_rm = _transpose_tables(W_in.T, W_out.T)

    centers_r = centers.reshape(NW, BPW, SPB)
    pos_r = pos.reshape(NW, BPW, SPB)
    neg_r = neg.reshape(NW, BPW * 5, D)

    psc, nsc = _sc_scores(centers_r, pos_r, neg_r, W_in_rm, W_out_rm)

    ps2 = psc.reshape(B // 128, 128)
    ns2 = nsc.reshape(B * K // 128, 128)
    loss = pl.pallas_call(
        _tc_loss_kernel,
        out_shape=jax.ShapeDtypeStruct((1, 1), jnp.float32),
    )(ps2, ns2)
    return loss.reshape(())
